# baseline (speedup 1.0000x reference)
import functools

import jax
import jax.numpy as jnp
from jax import lax
from jax.experimental import pallas as pl
from jax.experimental.pallas import tpu as pltpu
from jax.experimental.pallas import tpu_sc as plsc

NC, NS, LANES = 2, 16, 16
NW = NC * NS
EB = 96
NB = 108
E_PAD = NW * NB * EB
N_NODES = 10000
RPT = 632
RPT_LAST = N_NODES - 15 * RPT
FC = 128


def _sc_agg(h, e3, ew3, zeros):
    mesh = plsc.VectorSubcoreMesh(
        core_axis_name="c", subcore_axis_name="s", num_cores=NC, num_subcores=NS)

    @functools.partial(
        pl.kernel,
        out_type=jax.ShapeDtypeStruct((NC, N_NODES, FC), jnp.float32),
        mesh=mesh,
        scratch_types=[
            pltpu.VMEM((6, 2, EB), jnp.int32),
            pltpu.VMEM((6, EB), jnp.float32),
            pltpu.VMEM((3, EB, FC), jnp.float32),
            pltpu.VMEM_SHARED((N_NODES, FC), jnp.float32),
            pltpu.SemaphoreType.DMA,
            pltpu.SemaphoreType.DMA,
            pltpu.SemaphoreType.DMA,
        ],
    )
    def k(h_hbm, e3_hbm, ew_hbm, z_hbm, out_hbm, e3_v, ew_v, rows_v, acc_sh,
          gsem, ssem, isem):
        cid = lax.axis_index("c")
        sid = lax.axis_index("s")
        wid = cid * NS + sid
        r0 = sid * RPT

        @pl.when(sid < NS - 1)
        def _():
            pltpu.sync_copy(z_hbm.at[pl.ds(r0, RPT)], acc_sh.at[pl.ds(r0, RPT)])

        @pl.when(sid == NS - 1)
        def _():
            pltpu.sync_copy(z_hbm.at[pl.ds(15 * RPT, RPT_LAST)],
                            acc_sh.at[pl.ds(15 * RPT, RPT_LAST)])

        def fetch_idx(b, ib):
            pltpu.sync_copy(e3_hbm.at[wid, b], e3_v.at[ib])
            pltpu.sync_copy(ew_hbm.at[wid, b], ew_v.at[ib])

        def idx_start(b, ib):
            pltpu.async_copy(e3_hbm.at[wid, b], e3_v.at[ib], isem)
            pltpu.async_copy(ew_hbm.at[wid, b], ew_v.at[ib], isem)

        def idx_wait(b, ib):
            pltpu.make_async_copy(e3_hbm.at[wid, b], e3_v.at[ib], isem).wait()
            pltpu.make_async_copy(ew_hbm.at[wid, b], ew_v.at[ib], isem).wait()

        def gather(b, rb, ib):
            return pltpu.make_async_copy(
                h_hbm.at[e3_v.at[ib, 0]], rows_v.at[rb], gsem)

        def scale(rb, ib):
            def e16_body(e16, c2):
                wv = ew_v[ib, pl.ds(e16 * LANES, LANES)]
                for i in range(LANES):
                    e = e16 * LANES + i
                    w16 = jnp.broadcast_to(wv[i], (LANES,))
                    for j in range(FC // LANES):
                        sl = pl.ds(j * LANES, LANES)
                        rows_v[rb, e, sl] = rows_v[rb, e, sl] * w16
                return c2
            lax.fori_loop(0, EB // LANES, e16_body, 0)

        def scatter_start(rb, ib):
            pltpu.async_copy(rows_v.at[rb], acc_sh.at[e3_v.at[ib, 1]],
                             ssem, add=True)

        def scatter_wait(rb, ib):
            pltpu.make_async_copy(rows_v.at[rb],
                                  acc_sh.at[e3_v.at[ib, 1]], ssem).wait()

        plsc.subcore_barrier()
        fetch_idx(0, 0)
        fetch_idx(1, 1)
        fetch_idx(2, 2)
        gather(0, 0, 0).start()
        gather(1, 1, 1).start()
        sixth = NB // 6

        def group_body(g, carry):
            for i in range(6):
                b = 6 * g + i
                rb = i % 3
                ib = i
                rb1 = (i + 2) % 3
                ib1 = (i + 5) % 6
                ib2 = (i + 2) % 6
                ib3 = (i + 3) % 6

                @pl.when((b >= 1) & (b + 2 < NB))
                def _():
                    idx_wait(b + 2, ib2)

                gather(b, rb, ib).wait()

                @pl.when(b >= 1)
                def _():
                    scatter_wait(rb1, ib1)

                @pl.when(b + 2 < NB)
                def _():
                    gather(b + 2, rb1, ib2).start()

                @pl.when(b + 3 < NB)
                def _():
                    idx_start(b + 3, ib3)

                scale(rb, ib)
                scatter_start(rb, ib)
            return carry

        lax.fori_loop(0, sixth, group_body, 0)
        scatter_wait((NB - 1) % 3, (NB - 1) % 6)
        plsc.subcore_barrier()

        @pl.when(sid < NS - 1)
        def _():
            pltpu.sync_copy(acc_sh.at[pl.ds(r0, RPT)],
                            out_hbm.at[cid, pl.ds(r0, RPT)])

        @pl.when(sid == NS - 1)
        def _():
            pltpu.sync_copy(acc_sh.at[pl.ds(15 * RPT, RPT_LAST)],
                            out_hbm.at[cid, pl.ds(15 * RPT, RPT_LAST)])

    return k(h, e3, ew3, zeros)


def _dense2(parts, h, wa_t, wb_t, b2d, relu, bn=400):
    n, fin = h.shape
    fout = wa_t.shape[1]

    def body(p_ref, h_ref, wa_ref, wb_ref, b_ref, o_ref):
        agg = p_ref[0] + p_ref[1]
        z = jnp.dot(agg, wa_ref[...], preferred_element_type=jnp.float32,
                    precision=lax.Precision.HIGHEST)
        z = z + jnp.dot(h_ref[...], wb_ref[...], preferred_element_type=jnp.float32,
                        precision=lax.Precision.HIGHEST)
        z = z + b_ref[...]
        o_ref[...] = jnp.maximum(z, 0.0) if relu else z

    return pl.pallas_call(
        body,
        grid=(n // bn,),
        in_specs=[
            pl.BlockSpec((2, bn, fin), lambda i: (0, i, 0)),
            pl.BlockSpec((bn, fin), lambda i: (i, 0)),
            pl.BlockSpec((fin, fout), lambda i: (0, 0)),
            pl.BlockSpec((fin, fout), lambda i: (0, 0)),
            pl.BlockSpec((1, fout), lambda i: (0, 0)),
        ],
        out_specs=pl.BlockSpec((bn, fout), lambda i: (i, 0)),
        out_shape=jax.ShapeDtypeStruct((n, fout), jnp.float32),
    )(parts, h, wa_t, wb_t, b2d)


def _matmul(h, w_t, bn=400):
    n, fin = h.shape
    fout = w_t.shape[1]

    def body(h_ref, w_ref, o_ref):
        o_ref[...] = jnp.dot(h_ref[...], w_ref[...],
                             preferred_element_type=jnp.float32,
                             precision=lax.Precision.HIGHEST)

    return pl.pallas_call(
        body,
        grid=(n // bn,),
        in_specs=[
            pl.BlockSpec((bn, fin), lambda i: (i, 0)),
            pl.BlockSpec((fin, fout), lambda i: (0, 0)),
        ],
        out_specs=pl.BlockSpec((bn, fout), lambda i: (i, 0)),
        out_shape=jax.ShapeDtypeStruct((n, fout), jnp.float32),
    )(h, w_t)


def _final(parts, h, w_t, b2d, bn=400):
    n, fin = h.shape
    fout = w_t.shape[1]

    def body(p_ref, h_ref, w_ref, b_ref, o_ref):
        z = jnp.dot(h_ref[...], w_ref[...], preferred_element_type=jnp.float32,
                    precision=lax.Precision.HIGHEST)
        o_ref[...] = p_ref[0] + p_ref[1] + z + b_ref[...]

    return pl.pallas_call(
        body,
        grid=(n // bn,),
        in_specs=[
            pl.BlockSpec((2, bn, fout), lambda i: (0, i, 0)),
            pl.BlockSpec((bn, fin), lambda i: (i, 0)),
            pl.BlockSpec((fin, fout), lambda i: (0, 0)),
            pl.BlockSpec((1, fout), lambda i: (0, 0)),
        ],
        out_specs=pl.BlockSpec((bn, fout), lambda i: (i, 0)),
        out_shape=jax.ShapeDtypeStruct((n, fout), jnp.float32),
    )(parts, h, w_t, b2d)


def kernel(x, edge_index, edge_weight, W1_rel, b1, W1_root,
           W2_rel, b2, W2_root, W3_rel, b3, W3_root):
    src = edge_index[0]
    dst = edge_index[1]
    pad = E_PAD - src.shape[0]
    src_r = jnp.concatenate([src, jnp.zeros((pad,), jnp.int32)]).reshape(NW, NB, EB)
    dst_r = jnp.concatenate([dst, jnp.zeros((pad,), jnp.int32)]).reshape(NW, NB, EB)
    ew3 = jnp.concatenate([edge_weight, jnp.zeros((pad,), jnp.float32)]).reshape(NW, NB, EB)
    e3 = jnp.stack([src_r, dst_r], axis=2)
    zeros = jnp.zeros((N_NODES, FC), jnp.float32)

    p1 = _sc_agg(x, e3, ew3, zeros)
    h1 = _dense2(p1, x, W1_rel.T, W1_root.T, b1.reshape(1, -1), relu=True)

    p2a = _sc_agg(h1[:, :FC], e3, ew3, zeros)
    p2b = _sc_agg(h1[:, FC:], e3, ew3, zeros)
    parts2 = jnp.concatenate([p2a, p2b], axis=2)
    h2 = _dense2(parts2, h1, W2_rel.T, W2_root.T, b2.reshape(1, -1), relu=True)

    y3 = _matmul(h2, W3_rel.T)
    p3 = _sc_agg(y3, e3, ew3, zeros)
    return _final(p3, h2, W3_root.T, b3.reshape(1, -1))


# speedup vs baseline: 2.0399x; 2.0399x over previous
import functools

import jax
import jax.numpy as jnp
from jax import lax
from jax.experimental import pallas as pl
from jax.experimental.pallas import tpu as pltpu
from jax.experimental.pallas import tpu_sc as plsc

NC, NS, LANES = 2, 16, 16
NW = NC * NS
EB = 96
NB = 105
E_PAD = NW * NB * EB
N_NODES = 10000
RPT = 632
RPT_LAST = N_NODES - 15 * RPT
FC = 128


def _sc_agg(h, e3, ew3, zeros):
    mesh = plsc.VectorSubcoreMesh(
        core_axis_name="c", subcore_axis_name="s", num_cores=NC, num_subcores=NS)

    @functools.partial(
        pl.kernel,
        out_type=jax.ShapeDtypeStruct((NC, N_NODES, FC), jnp.float32),
        mesh=mesh,
        scratch_types=[
            pltpu.VMEM((3, 2, EB), jnp.int32),
            pltpu.VMEM((3, EB), jnp.float32),
            pltpu.VMEM((3, EB, FC), jnp.float32),
            pltpu.VMEM_SHARED((N_NODES, FC), jnp.float32),
            pltpu.SemaphoreType.DMA,
            pltpu.SemaphoreType.DMA,
        ],
    )
    def k(h_hbm, e3_hbm, ew_hbm, z_hbm, out_hbm, e3_v, ew_v, rows_v, acc_sh,
          gsem, ssem):
        cid = lax.axis_index("c")
        sid = lax.axis_index("s")
        wid = cid * NS + sid
        r0 = sid * RPT

        @pl.when(sid < NS - 1)
        def _():
            pltpu.sync_copy(z_hbm.at[pl.ds(r0, RPT)], acc_sh.at[pl.ds(r0, RPT)])

        @pl.when(sid == NS - 1)
        def _():
            pltpu.sync_copy(z_hbm.at[pl.ds(15 * RPT, RPT_LAST)],
                            acc_sh.at[pl.ds(15 * RPT, RPT_LAST)])

        def fetch_idx(b, buf):
            pltpu.sync_copy(e3_hbm.at[wid, b], e3_v.at[buf])
            pltpu.sync_copy(ew_hbm.at[wid, b], ew_v.at[buf])

        def gather(b, buf):
            return pltpu.make_async_copy(
                h_hbm.at[e3_v.at[buf, 0]], rows_v.at[buf], gsem)

        def scale(buf):
            def e16_body(e16, c2):
                wv = ew_v[buf, pl.ds(e16 * LANES, LANES)]
                for i in range(LANES):
                    e = e16 * LANES + i
                    w16 = jnp.broadcast_to(wv[i], (LANES,))
                    for j in range(FC // LANES):
                        sl = pl.ds(j * LANES, LANES)
                        rows_v[buf, e, sl] = rows_v[buf, e, sl] * w16
                return c2
            lax.fori_loop(0, EB // LANES, e16_body, 0)

        def scatter_start(buf):
            pltpu.async_copy(rows_v.at[buf], acc_sh.at[e3_v.at[buf, 1]],
                             ssem, add=True)

        def scatter_wait(buf):
            pltpu.make_async_copy(rows_v.at[buf],
                                  acc_sh.at[e3_v.at[buf, 1]], ssem).wait()

        plsc.subcore_barrier()
        fetch_idx(0, 0)
        gather(0, 0).start()
        fetch_idx(1, 1)
        gather(1, 1).start()
        third = NB // 3

        def group_body(g, carry):
            for buf in (0, 1, 2):
                b = 3 * g + buf
                nxt = (buf + 2) % 3
                gather(b, buf).wait()

                @pl.when(b >= 1)
                def _():
                    scatter_wait(nxt)

                @pl.when(b + 2 < NB)
                def _():
                    fetch_idx(b + 2, nxt)
                    gather(b + 2, nxt).start()

                scale(buf)
                scatter_start(buf)
            return carry

        lax.fori_loop(0, third, group_body, 0)
        scatter_wait((NB - 1) % 3)
        plsc.subcore_barrier()

        @pl.when(sid < NS - 1)
        def _():
            pltpu.sync_copy(acc_sh.at[pl.ds(r0, RPT)],
                            out_hbm.at[cid, pl.ds(r0, RPT)])

        @pl.when(sid == NS - 1)
        def _():
            pltpu.sync_copy(acc_sh.at[pl.ds(15 * RPT, RPT_LAST)],
                            out_hbm.at[cid, pl.ds(15 * RPT, RPT_LAST)])

    return k(h, e3, ew3, zeros)


def _dense2(parts, h, wa_t, wb_t, b2d, relu, bn=400):
    n, fin = h.shape
    fout = wa_t.shape[1]

    def body(p_ref, h_ref, wa_ref, wb_ref, b_ref, o_ref):
        agg = p_ref[0] + p_ref[1]
        z = jnp.dot(agg, wa_ref[...], preferred_element_type=jnp.float32,
                    precision=lax.Precision.HIGHEST)
        z = z + jnp.dot(h_ref[...], wb_ref[...], preferred_element_type=jnp.float32,
                        precision=lax.Precision.HIGHEST)
        z = z + b_ref[...]
        o_ref[...] = jnp.maximum(z, 0.0) if relu else z

    return pl.pallas_call(
        body,
        grid=(n // bn,),
        in_specs=[
            pl.BlockSpec((2, bn, fin), lambda i: (0, i, 0)),
            pl.BlockSpec((bn, fin), lambda i: (i, 0)),
            pl.BlockSpec((fin, fout), lambda i: (0, 0)),
            pl.BlockSpec((fin, fout), lambda i: (0, 0)),
            pl.BlockSpec((1, fout), lambda i: (0, 0)),
        ],
        out_specs=pl.BlockSpec((bn, fout), lambda i: (i, 0)),
        out_shape=jax.ShapeDtypeStruct((n, fout), jnp.float32),
    )(parts, h, wa_t, wb_t, b2d)


def _matmul(h, w_t, bn=400):
    n, fin = h.shape
    fout = w_t.shape[1]

    def body(h_ref, w_ref, o_ref):
        o_ref[...] = jnp.dot(h_ref[...], w_ref[...],
                             preferred_element_type=jnp.float32,
                             precision=lax.Precision.HIGHEST)

    return pl.pallas_call(
        body,
        grid=(n // bn,),
        in_specs=[
            pl.BlockSpec((bn, fin), lambda i: (i, 0)),
            pl.BlockSpec((fin, fout), lambda i: (0, 0)),
        ],
        out_specs=pl.BlockSpec((bn, fout), lambda i: (i, 0)),
        out_shape=jax.ShapeDtypeStruct((n, fout), jnp.float32),
    )(h, w_t)


def _final(parts, h, w_t, b2d, bn=400):
    n, fin = h.shape
    fout = w_t.shape[1]

    def body(p_ref, h_ref, w_ref, b_ref, o_ref):
        z = jnp.dot(h_ref[...], w_ref[...], preferred_element_type=jnp.float32,
                    precision=lax.Precision.HIGHEST)
        o_ref[...] = p_ref[0] + p_ref[1] + z + b_ref[...]

    return pl.pallas_call(
        body,
        grid=(n // bn,),
        in_specs=[
            pl.BlockSpec((2, bn, fout), lambda i: (0, i, 0)),
            pl.BlockSpec((bn, fin), lambda i: (i, 0)),
            pl.BlockSpec((fin, fout), lambda i: (0, 0)),
            pl.BlockSpec((1, fout), lambda i: (0, 0)),
        ],
        out_specs=pl.BlockSpec((bn, fout), lambda i: (i, 0)),
        out_shape=jax.ShapeDtypeStruct((n, fout), jnp.float32),
    )(parts, h, w_t, b2d)


def kernel(x, edge_index, edge_weight, W1_rel, b1, W1_root,
           W2_rel, b2, W2_root, W3_rel, b3, W3_root):
    src = edge_index[0]
    dst = edge_index[1]
    pad = E_PAD - src.shape[0]
    src_r = jnp.concatenate([src, jnp.zeros((pad,), jnp.int32)]).reshape(NW, NB, EB)
    dst_r = jnp.concatenate([dst, jnp.zeros((pad,), jnp.int32)]).reshape(NW, NB, EB)
    ew3 = jnp.concatenate([edge_weight, jnp.zeros((pad,), jnp.float32)]).reshape(NW, NB, EB)
    e3 = jnp.stack([src_r, dst_r], axis=2)
    zeros = jnp.zeros((N_NODES, FC), jnp.float32)

    p1 = _sc_agg(x, e3, ew3, zeros)
    h1 = _dense2(p1, x, W1_rel.T, W1_root.T, b1.reshape(1, -1), relu=True)

    p2a = _sc_agg(h1[:, :FC], e3, ew3, zeros)
    p2b = _sc_agg(h1[:, FC:], e3, ew3, zeros)
    parts2 = jnp.concatenate([p2a, p2b], axis=2)
    h2 = _dense2(parts2, h1, W2_rel.T, W2_root.T, b2.reshape(1, -1), relu=True)

    y3 = _matmul(h2, W3_rel.T)
    p3 = _sc_agg(y3, e3, ew3, zeros)
    return _final(p3, h2, W3_root.T, b3.reshape(1, -1))
